# baseline (speedup 1.0000x reference)
import jax
import jax.numpy as jnp
from jax.experimental import pallas as pl
from jax.experimental.pallas import tpu as pltpu
from jax.experimental.pallas import tpu_sc as plsc

_HIDDEN = 768
_NUM_MOD = 64
_N_TOK = 2048
_TILE = 128
_NBUF = 4


def _sc_seg_offsets(mm):
    mesh = plsc.ScalarSubcoreMesh(axis_name="core")
    n_cores = mesh.num_cores
    per_core = -(-_NUM_MOD // n_cores)

    @pl.kernel(
        out_type=jax.ShapeDtypeStruct((_NUM_MOD, 2), jnp.int32),
        mesh=mesh,
        scratch_types=[
            pltpu.SMEM((_N_TOK,), jnp.int32),
            pltpu.SMEM((_NUM_MOD, 2), jnp.int32),
            pltpu.SemaphoreType.DMA,
        ],
    )
    def sc_meta(mm_hbm, se_hbm, mm_s, se_s, sem):
        c = jax.lax.axis_index("core")
        e_lo = c * per_core
        e_hi = jnp.minimum(e_lo + per_core, _NUM_MOD)

        pltpu.async_copy(mm_hbm, mm_s, sem).wait()

        def count_le(v):
            def step_fn(_, carry):
                cnt, step = carry
                idx = jnp.minimum(cnt + step - 1, _N_TOK - 1)
                ok = jnp.logical_and(cnt + step <= _N_TOK, mm_s[idx] <= v)
                return cnt + jnp.where(ok, step, 0), step // 2

            cnt, _ = jax.lax.fori_loop(0, 12, step_fn, (0, _N_TOK))
            return cnt

        @pl.loop(e_lo, e_hi)
        def _(e):
            se_s[e, 0] = count_le(e - 1)
            se_s[e, 1] = count_le(e)

        pltpu.async_copy(
            se_s.at[pl.ds(e_lo, per_core), :],
            se_hbm.at[pl.ds(e_lo, per_core), :],
            sem,
        ).wait()

    return sc_meta(mm)


def _gmm_kernel(se_ref, x_ref, w_hbm, nw_ref, out_ref, wbuf, sems):
    def start_copy(e, slot):
        pltpu.make_async_copy(w_hbm.at[e], wbuf.at[slot], sems.at[slot]).start()

    for k in range(_NBUF - 1):
        start_copy(k, k)

    def process_expert(e):
        nxt = e + _NBUF - 1

        @pl.when(nxt < _NUM_MOD)
        def _():
            start_copy(nxt, jax.lax.rem(nxt, _NBUF))

        slot = jax.lax.rem(e, _NBUF)
        pltpu.make_async_copy(
            w_hbm.at[e], wbuf.at[slot], sems.at[slot]
        ).wait()

        seg_lo = se_ref[e, 0]
        seg_hi = se_ref[e, 1]
        scale = nw_ref[e] + 1.0

        def tile_step(t, _):
            r0 = t * _TILE
            xs = x_ref[pl.ds(r0, _TILE), :]
            normed = (xs * scale).astype(jnp.bfloat16)
            y = jax.lax.dot_general(
                normed,
                wbuf[slot],
                dimension_numbers=(((1,), (1,)), ((), ())),
                preferred_element_type=jnp.float32,
            )
            rows = r0 + jax.lax.broadcasted_iota(jnp.int32, (_TILE, 1), 0)
            mask = jnp.logical_and(rows >= seg_lo, rows < seg_hi)
            out_ref[pl.ds(r0, _TILE), :] = jnp.where(
                mask, y, out_ref[pl.ds(r0, _TILE), :]
            )
            return 0

        jax.lax.fori_loop(
            seg_lo // _TILE, (seg_hi + _TILE - 1) // _TILE, tile_step, 0
        )

    def expert_pair_step(i, _):
        process_expert(2 * i)
        process_expert(2 * i + 1)
        return 0

    jax.lax.fori_loop(0, _NUM_MOD // 2, expert_pair_step, 0)


def kernel(x, modality_mapping, W, norm_w):
    mm = modality_mapping.astype(jnp.int32)
    se = _sc_seg_offsets(mm)

    grid_spec = pltpu.PrefetchScalarGridSpec(
        num_scalar_prefetch=1,
        grid=(1,),
        in_specs=[
            pl.BlockSpec((_N_TOK, _HIDDEN), lambda g, s: (0, 0)),
            pl.BlockSpec(memory_space=pl.ANY),
            pl.BlockSpec((_NUM_MOD, _HIDDEN), lambda g, s: (0, 0)),
        ],
        out_specs=pl.BlockSpec((_N_TOK, _HIDDEN), lambda g, s: (0, 0)),
        scratch_shapes=[
            pltpu.VMEM((_NBUF, _HIDDEN, _HIDDEN), jnp.bfloat16),
            pltpu.SemaphoreType.DMA((_NBUF,)),
        ],
    )
    return pl.pallas_call(
        _gmm_kernel,
        grid_spec=grid_spec,
        out_shape=jax.ShapeDtypeStruct((_N_TOK, _HIDDEN), jnp.float32),
        compiler_params=pltpu.CompilerParams(
            dimension_semantics=("arbitrary",)
        ),
    )(se, x, W, norm_w)


# speedup vs baseline: 1.0348x; 1.0348x over previous
import jax
import jax.numpy as jnp
from jax.experimental import pallas as pl
from jax.experimental.pallas import tpu as pltpu
from jax.experimental.pallas import tpu_sc as plsc

_HIDDEN = 768
_NUM_MOD = 64
_N_TOK = 2048
_TILE = 128
_NBUF = 4


def _sc_seg_offsets(mm):
    mesh = plsc.ScalarSubcoreMesh(axis_name="core")
    n_cores = mesh.num_cores
    per_core = -(-_NUM_MOD // n_cores)

    @pl.kernel(
        out_type=jax.ShapeDtypeStruct((_NUM_MOD, 2), jnp.int32),
        mesh=mesh,
        scratch_types=[
            pltpu.SMEM((_N_TOK,), jnp.int32),
            pltpu.SMEM((_NUM_MOD, 2), jnp.int32),
            pltpu.SemaphoreType.DMA,
        ],
    )
    def sc_meta(mm_hbm, se_hbm, mm_s, se_s, sem):
        c = jax.lax.axis_index("core")
        e_lo = c * per_core
        e_hi = jnp.minimum(e_lo + per_core, _NUM_MOD)

        pltpu.async_copy(mm_hbm, mm_s, sem).wait()

        def count_le(v):
            def step_fn(_, carry):
                cnt, step = carry
                idx = jnp.minimum(cnt + step - 1, _N_TOK - 1)
                ok = jnp.logical_and(cnt + step <= _N_TOK, mm_s[idx] <= v)
                return cnt + jnp.where(ok, step, 0), step // 2

            cnt, _ = jax.lax.fori_loop(0, 12, step_fn, (0, _N_TOK))
            return cnt

        se_s[e_lo, 0] = count_le(e_lo - 1)

        @pl.loop(e_lo, e_hi)
        def _(e):
            end_e = count_le(e)
            se_s[e, 1] = end_e

            @pl.when(e + 1 < e_hi)
            def _():
                se_s[e + 1, 0] = end_e

        pltpu.async_copy(
            se_s.at[pl.ds(e_lo, per_core), :],
            se_hbm.at[pl.ds(e_lo, per_core), :],
            sem,
        ).wait()

    return sc_meta(mm)


def _gmm_kernel(se_ref, x_ref, w_hbm, nw_ref, out_ref, wbuf, sems):
    def start_copy(e, slot):
        pltpu.make_async_copy(w_hbm.at[e], wbuf.at[slot], sems.at[slot]).start()

    for k in range(_NBUF - 1):
        start_copy(k, k)

    def process_expert(e):
        nxt = e + _NBUF - 1

        @pl.when(nxt < _NUM_MOD)
        def _():
            start_copy(nxt, jax.lax.rem(nxt, _NBUF))

        slot = jax.lax.rem(e, _NBUF)
        pltpu.make_async_copy(
            w_hbm.at[e], wbuf.at[slot], sems.at[slot]
        ).wait()

        seg_lo = se_ref[e, 0]
        seg_hi = se_ref[e, 1]
        scale = nw_ref[e] + 1.0

        def tile_step(t, _):
            r0 = t * _TILE
            xs = x_ref[pl.ds(r0, _TILE), :]
            normed = (xs * scale).astype(jnp.bfloat16)
            y = jax.lax.dot_general(
                normed,
                wbuf[slot],
                dimension_numbers=(((1,), (1,)), ((), ())),
                preferred_element_type=jnp.float32,
            )
            rows = r0 + jax.lax.broadcasted_iota(jnp.int32, (_TILE, 1), 0)
            mask = jnp.logical_and(rows >= seg_lo, rows < seg_hi)
            out_ref[pl.ds(r0, _TILE), :] = jnp.where(
                mask, y, out_ref[pl.ds(r0, _TILE), :]
            )
            return 0

        jax.lax.fori_loop(
            seg_lo // _TILE, (seg_hi + _TILE - 1) // _TILE, tile_step, 0
        )

    def expert_pair_step(i, _):
        process_expert(2 * i)
        process_expert(2 * i + 1)
        return 0

    jax.lax.fori_loop(0, _NUM_MOD // 2, expert_pair_step, 0)


def kernel(x, modality_mapping, W, norm_w):
    mm = modality_mapping.astype(jnp.int32)
    se = _sc_seg_offsets(mm)

    grid_spec = pltpu.PrefetchScalarGridSpec(
        num_scalar_prefetch=1,
        grid=(1,),
        in_specs=[
            pl.BlockSpec((_N_TOK, _HIDDEN), lambda g, s: (0, 0)),
            pl.BlockSpec(memory_space=pl.ANY),
            pl.BlockSpec((_NUM_MOD, _HIDDEN), lambda g, s: (0, 0)),
        ],
        out_specs=pl.BlockSpec((_N_TOK, _HIDDEN), lambda g, s: (0, 0)),
        scratch_shapes=[
            pltpu.VMEM((_NBUF, _HIDDEN, _HIDDEN), jnp.bfloat16),
            pltpu.SemaphoreType.DMA((_NBUF,)),
        ],
    )
    return pl.pallas_call(
        _gmm_kernel,
        grid_spec=grid_spec,
        out_shape=jax.ShapeDtypeStruct((_N_TOK, _HIDDEN), jnp.float32),
        compiler_params=pltpu.CompilerParams(
            dimension_semantics=("arbitrary",)
        ),
    )(se, x, W, norm_w)


# speedup vs baseline: 1.0359x; 1.0011x over previous
import jax
import jax.numpy as jnp
from jax.experimental import pallas as pl
from jax.experimental.pallas import tpu as pltpu
from jax.experimental.pallas import tpu_sc as plsc

_HIDDEN = 768
_NUM_MOD = 64
_N_TOK = 2048
_TILE = 128
_NBUF = 6


def _sc_seg_offsets(mm):
    mesh = plsc.ScalarSubcoreMesh(axis_name="core")
    n_cores = mesh.num_cores
    per_core = -(-_NUM_MOD // n_cores)

    @pl.kernel(
        out_type=jax.ShapeDtypeStruct((_NUM_MOD, 2), jnp.int32),
        mesh=mesh,
        scratch_types=[
            pltpu.SMEM((_N_TOK,), jnp.int32),
            pltpu.SMEM((_NUM_MOD, 2), jnp.int32),
            pltpu.SemaphoreType.DMA,
        ],
    )
    def sc_meta(mm_hbm, se_hbm, mm_s, se_s, sem):
        c = jax.lax.axis_index("core")
        e_lo = c * per_core
        e_hi = jnp.minimum(e_lo + per_core, _NUM_MOD)

        pltpu.async_copy(mm_hbm, mm_s, sem).wait()

        def count_le(v):
            def step_fn(_, carry):
                cnt, step = carry
                idx = jnp.minimum(cnt + step - 1, _N_TOK - 1)
                ok = jnp.logical_and(cnt + step <= _N_TOK, mm_s[idx] <= v)
                return cnt + jnp.where(ok, step, 0), step // 2

            cnt, _ = jax.lax.fori_loop(0, 12, step_fn, (0, _N_TOK))
            return cnt

        se_s[e_lo, 0] = count_le(e_lo - 1)

        @pl.loop(e_lo, e_hi)
        def _(e):
            end_e = count_le(e)
            se_s[e, 1] = end_e

            @pl.when(e + 1 < e_hi)
            def _():
                se_s[e + 1, 0] = end_e

        pltpu.async_copy(
            se_s.at[pl.ds(e_lo, per_core), :],
            se_hbm.at[pl.ds(e_lo, per_core), :],
            sem,
        ).wait()

    return sc_meta(mm)


def _gmm_kernel(se_ref, x_ref, w_hbm, nw_ref, out_ref, wbuf, sems):
    def start_copy(e, slot):
        pltpu.make_async_copy(w_hbm.at[e], wbuf.at[slot], sems.at[slot]).start()

    for k in range(_NBUF - 1):
        start_copy(k, k)

    def process_expert(e):
        nxt = e + _NBUF - 1

        @pl.when(nxt < _NUM_MOD)
        def _():
            start_copy(nxt, jax.lax.rem(nxt, _NBUF))

        slot = jax.lax.rem(e, _NBUF)
        pltpu.make_async_copy(
            w_hbm.at[e], wbuf.at[slot], sems.at[slot]
        ).wait()

        seg_lo = se_ref[e, 0]
        seg_hi = se_ref[e, 1]
        scale = nw_ref[e] + 1.0

        def tile_step(t, _):
            r0 = t * _TILE
            xs = x_ref[pl.ds(r0, _TILE), :]
            normed = (xs * scale).astype(jnp.bfloat16)
            y = jax.lax.dot_general(
                normed,
                wbuf[slot],
                dimension_numbers=(((1,), (1,)), ((), ())),
                preferred_element_type=jnp.float32,
            )
            rows = r0 + jax.lax.broadcasted_iota(jnp.int32, (_TILE, 1), 0)
            mask = jnp.logical_and(rows >= seg_lo, rows < seg_hi)
            out_ref[pl.ds(r0, _TILE), :] = jnp.where(
                mask, y, out_ref[pl.ds(r0, _TILE), :]
            )
            return 0

        jax.lax.fori_loop(
            seg_lo // _TILE, (seg_hi + _TILE - 1) // _TILE, tile_step, 0
        )

    def expert_pair_step(i, _):
        process_expert(2 * i)
        process_expert(2 * i + 1)
        return 0

    jax.lax.fori_loop(0, _NUM_MOD // 2, expert_pair_step, 0)


def kernel(x, modality_mapping, W, norm_w):
    mm = modality_mapping.astype(jnp.int32)
    se = _sc_seg_offsets(mm)

    grid_spec = pltpu.PrefetchScalarGridSpec(
        num_scalar_prefetch=1,
        grid=(1,),
        in_specs=[
            pl.BlockSpec((_N_TOK, _HIDDEN), lambda g, s: (0, 0)),
            pl.BlockSpec(memory_space=pl.ANY),
            pl.BlockSpec((_NUM_MOD, _HIDDEN), lambda g, s: (0, 0)),
        ],
        out_specs=pl.BlockSpec((_N_TOK, _HIDDEN), lambda g, s: (0, 0)),
        scratch_shapes=[
            pltpu.VMEM((_NBUF, _HIDDEN, _HIDDEN), jnp.bfloat16),
            pltpu.SemaphoreType.DMA((_NBUF,)),
        ],
    )
    return pl.pallas_call(
        _gmm_kernel,
        grid_spec=grid_spec,
        out_shape=jax.ShapeDtypeStruct((_N_TOK, _HIDDEN), jnp.float32),
        compiler_params=pltpu.CompilerParams(
            dimension_semantics=("arbitrary",)
        ),
    )(se, x, W, norm_w)
